# baseline (speedup 1.0000x reference)
import functools

import jax
import jax.numpy as jnp
from jax import lax
from jax.experimental import pallas as pl
from jax.experimental.pallas import tpu as pltpu
from jax.experimental.pallas import tpu_sc as plsc

B = 4
F = 192
HW = 224 * 224
S = 512
L = 16
NC, NS = 2, 16
NW = NC * NS
TPB = NW // B
FPT = F // TPB
K = 6
GRP = FPT // K
RH = 8
W = 224
CH = RH * W
NCH = 224 // RH
ACC = S * L + L


def _body(img_hbm, idx_hbm, out_hbm, addr_v,
          a0, a1, a2, a3, a4, a5,
          buf0, buf1, out_v, sem0, sem1):
    accs = (a0, a1, a2, a3, a4, a5)
    wid = lax.axis_index("s") * NC + lax.axis_index("c")
    b = wid // TPB
    f0 = (wid % TPB) * FPT
    lane = lax.broadcasted_iota(jnp.int32, (L,), 0)

    pltpu.sync_copy(idx_hbm.at[b], addr_v)

    def mk_addr(i, c):
        v = addr_v[pl.ds(i * L, L)]
        v = jnp.minimum(v, S)
        addr_v[pl.ds(i * L, L)] = jnp.where(
            v == 0, S * L + lane, (v - 1) * L + lane)
        return c

    lax.fori_loop(0, HW // L, mk_addr, 0)

    def src(g, c):
        return img_hbm.at[b, pl.ds(f0 + g * K, K), pl.ds(c * RH, RH), :]

    def process(cur, ro, carry):
        def ldstep(i):
            r = i // (W // L)
            w0 = (i % (W // L)) * L
            ad = addr_v[pl.ds(ro * W + i * L, L)]
            vs = [cur[k, r, pl.ds(w0, L)] for k in range(K)]
            return ad, vs

        ad0, vs0 = ldstep(0)
        gs0 = [plsc.load_gather(accs[k], [ad0]) for k in range(K)]
        ms0 = tuple(jnp.maximum(g, v) for g, v in zip(gs0, vs0))

        def scat(i, st):
            cc, ad_p, ms_p = st
            ad, vs = ldstep(i)
            for k in range(K):
                plsc.store_scatter(accs[k], [ad_p], ms_p[k])
            gs = [plsc.load_gather(accs[k], [ad]) for k in range(K)]
            ms = tuple(jnp.maximum(g, v) for g, v in zip(gs, vs))
            return (cc, ad, ms)

        cc, ad_l, ms_l = lax.fori_loop(1, CH // L, scat, (carry, ad0, ms0))
        for k in range(K):
            plsc.store_scatter(accs[k], [ad_l], ms_l[k])
        return cc

    perm_idx = tuple(jnp.bitwise_xor(lane, d) for d in (8, 4, 2, 1))
    lane_bit = tuple((lane & d) == 0 for d in (8, 4, 2, 1))

    pltpu.async_copy(src(0, 0), buf0, sem0)
    pltpu.async_copy(src(0, 1), buf1, sem1)

    def zero(i, cc):
        z = jnp.zeros((L,), jnp.float32)
        for acc in accs:
            acc[pl.ds(i * L, L)] = z
        return cc

    lax.fori_loop(0, ACC // L, zero, 0)

    def group_body(g, carry):
        def pair(c2, cc):
            c = c2 * 2
            pltpu.make_async_copy(src(g, c), buf0, sem0).wait()
            cc = process(buf0, c * RH, cc)
            pltpu.async_copy(src(g, c + 2), buf0, sem0)
            pltpu.make_async_copy(src(g, c + 1), buf1, sem1).wait()
            cc = process(buf1, (c + 1) * RH, cc)
            pltpu.async_copy(src(g, c + 3), buf1, sem1)
            return cc

        carry = lax.fori_loop(0, NCH // 2 - 1, pair, carry)

        gn = jnp.minimum(g + 1, GRP - 1)
        pltpu.make_async_copy(src(g, NCH - 2), buf0, sem0).wait()
        carry = process(buf0, (NCH - 2) * RH, carry)
        pltpu.async_copy(src(gn, 0), buf0, sem0)
        pltpu.make_async_copy(src(g, NCH - 1), buf1, sem1).wait()
        carry = process(buf1, (NCH - 1) * RH, carry)
        pltpu.async_copy(src(gn, 1), buf1, sem1)

        for k in range(K):
            def red(j, cc, *, acc=accs[k]):
                rows = [acc[pl.ds(j * (L * L) + i * L, L)]
                        for i in range(L)]
                for lvl, d in enumerate((8, 4, 2, 1)):
                    half = len(rows) // 2
                    nxt = []
                    for i in range(half):
                        va, vb = rows[i], rows[i + half]
                        pa = va.at[perm_idx[lvl]].get(
                            mode="promise_in_bounds")
                        pb = vb.at[perm_idx[lvl]].get(
                            mode="promise_in_bounds")
                        nxt.append(jnp.where(lane_bit[lvl],
                                             jnp.maximum(va, pa),
                                             jnp.maximum(vb, pb)))
                    rows = nxt
                out_v[pl.ds(j * L, L)] = rows[0]
                z = jnp.zeros((L,), jnp.float32)
                for i in range(L):
                    acc[pl.ds(j * (L * L) + i * L, L)] = z
                return cc

            carry = lax.fori_loop(0, S // L, red, carry)
            pltpu.sync_copy(out_v, out_hbm.at[b, f0 + g * K + k])
        return carry

    lax.fori_loop(0, GRP, group_body, 0)
    pltpu.make_async_copy(src(0, 0), buf0, sem0).wait()
    pltpu.make_async_copy(src(0, 1), buf1, sem1).wait()


@jax.jit
def _face_pool(img4, idx2):
    mesh = plsc.VectorSubcoreMesh(core_axis_name="c", subcore_axis_name="s")
    return pl.kernel(
        _body,
        out_type=jax.ShapeDtypeStruct((B, F, S), jnp.float32),
        mesh=mesh,
        compiler_params=pltpu.CompilerParams(needs_layout_passes=False),
        scratch_types=[
            pltpu.VMEM((HW,), jnp.int32),
            pltpu.VMEM((ACC,), jnp.float32),
            pltpu.VMEM((ACC,), jnp.float32),
            pltpu.VMEM((ACC,), jnp.float32),
            pltpu.VMEM((ACC,), jnp.float32),
            pltpu.VMEM((ACC,), jnp.float32),
            pltpu.VMEM((ACC,), jnp.float32),
            pltpu.VMEM((K, RH, W), jnp.float32),
            pltpu.VMEM((K, RH, W), jnp.float32),
            pltpu.VMEM((S,), jnp.float32),
            pltpu.SemaphoreType.DMA,
            pltpu.SemaphoreType.DMA,
        ],
    )(img4, idx2)


def kernel(img, index, max_index):
    del max_index
    b, f, h, w = img.shape
    idx2 = index.reshape(b, h * w).astype(jnp.int32)
    return _face_pool(img, idx2)


# speedup vs baseline: 1.0340x; 1.0340x over previous
import functools

import jax
import jax.numpy as jnp
from jax import lax
from jax.experimental import pallas as pl
from jax.experimental.pallas import tpu as pltpu
from jax.experimental.pallas import tpu_sc as plsc

B = 4
F = 192
HW = 224 * 224
S = 512
L = 16
NC, NS = 2, 16
NW = NC * NS
TPB = NW // B
FPT = F // TPB
K = 6
GRP = FPT // K
RH = 8
W = 224
CH = RH * W
NCH = 224 // RH
ACC = S * L + L


def _body(img_hbm, idx_hbm, out_hbm, addr_v,
          a0, a1, a2, a3, a4, a5,
          buf0, buf1, out_v, sem0, sem1):
    accs = (a0, a1, a2, a3, a4, a5)
    wid = lax.axis_index("s") * NC + lax.axis_index("c")
    b = wid // TPB
    f0 = (wid % TPB) * FPT
    lane = lax.broadcasted_iota(jnp.int32, (L,), 0)

    pltpu.sync_copy(idx_hbm.at[b], addr_v)

    def src(g, c):
        return img_hbm.at[b, pl.ds(f0 + g * K, K), pl.ds(c * RH, RH), :]

    def process(cur, ro, carry):
        def ldstep(i):
            r = i // (W // L)
            w0 = (i % (W // L)) * L
            v = addr_v[pl.ds(ro * W + i * L, L)]
            v = jnp.minimum(v, S)
            ad = jnp.where(v == 0, S * L + lane, (v - 1) * L + lane)
            vs = [cur[k, r, pl.ds(w0, L)] for k in range(K)]
            return ad, vs

        ad0, vs0 = ldstep(0)
        gs0 = [plsc.load_gather(accs[k], [ad0]) for k in range(K)]
        ms0 = tuple(jnp.maximum(g, v) for g, v in zip(gs0, vs0))

        def scat(i, st):
            cc, ad_p, ms_p = st
            ad, vs = ldstep(i)
            for k in range(K):
                plsc.store_scatter(accs[k], [ad_p], ms_p[k])
            gs = [plsc.load_gather(accs[k], [ad]) for k in range(K)]
            ms = tuple(jnp.maximum(g, v) for g, v in zip(gs, vs))
            return (cc, ad, ms)

        cc, ad_l, ms_l = lax.fori_loop(1, CH // L, scat, (carry, ad0, ms0))
        for k in range(K):
            plsc.store_scatter(accs[k], [ad_l], ms_l[k])
        return cc

    perm_idx = tuple(jnp.bitwise_xor(lane, d) for d in (8, 4, 2, 1))
    lane_bit = tuple((lane & d) == 0 for d in (8, 4, 2, 1))

    pltpu.async_copy(src(0, 0), buf0, sem0)
    pltpu.async_copy(src(0, 1), buf1, sem1)

    def group_body(g, carry):
        def zero(i, cc):
            z = jnp.zeros((L,), jnp.float32)
            for acc in accs:
                acc[pl.ds(i * L, L)] = z
            return cc

        carry = lax.fori_loop(0, ACC // L, zero, carry)

        def pair(c2, cc):
            c = c2 * 2
            pltpu.make_async_copy(src(g, c), buf0, sem0).wait()
            cc = process(buf0, c * RH, cc)
            pltpu.async_copy(src(g, c + 2), buf0, sem0)
            pltpu.make_async_copy(src(g, c + 1), buf1, sem1).wait()
            cc = process(buf1, (c + 1) * RH, cc)
            pltpu.async_copy(src(g, c + 3), buf1, sem1)
            return cc

        carry = lax.fori_loop(0, NCH // 2 - 1, pair, carry)

        gn = jnp.minimum(g + 1, GRP - 1)
        pltpu.make_async_copy(src(g, NCH - 2), buf0, sem0).wait()
        carry = process(buf0, (NCH - 2) * RH, carry)
        pltpu.async_copy(src(gn, 0), buf0, sem0)
        pltpu.make_async_copy(src(g, NCH - 1), buf1, sem1).wait()
        carry = process(buf1, (NCH - 1) * RH, carry)
        pltpu.async_copy(src(gn, 1), buf1, sem1)

        for k in range(K):
            def red(j, cc, *, acc=accs[k]):
                rows = [acc[pl.ds(j * (L * L) + i * L, L)]
                        for i in range(L)]
                for lvl, d in enumerate((8, 4, 2, 1)):
                    half = len(rows) // 2
                    nxt = []
                    for i in range(half):
                        va, vb = rows[i], rows[i + half]
                        pa = va.at[perm_idx[lvl]].get(
                            mode="promise_in_bounds")
                        pb = vb.at[perm_idx[lvl]].get(
                            mode="promise_in_bounds")
                        nxt.append(jnp.where(lane_bit[lvl],
                                             jnp.maximum(va, pa),
                                             jnp.maximum(vb, pb)))
                    rows = nxt
                out_v[pl.ds(j * L, L)] = rows[0]
                return cc

            carry = lax.fori_loop(0, S // L, red, carry)
            pltpu.sync_copy(out_v, out_hbm.at[b, f0 + g * K + k])
        return carry

    lax.fori_loop(0, GRP, group_body, 0)
    pltpu.make_async_copy(src(0, 0), buf0, sem0).wait()
    pltpu.make_async_copy(src(0, 1), buf1, sem1).wait()


@jax.jit
def _face_pool(img4, idx2):
    mesh = plsc.VectorSubcoreMesh(core_axis_name="c", subcore_axis_name="s")
    return pl.kernel(
        _body,
        out_type=jax.ShapeDtypeStruct((B, F, S), jnp.float32),
        mesh=mesh,
        compiler_params=pltpu.CompilerParams(needs_layout_passes=False),
        scratch_types=[
            pltpu.VMEM((HW,), jnp.int32),
            pltpu.VMEM((ACC,), jnp.float32),
            pltpu.VMEM((ACC,), jnp.float32),
            pltpu.VMEM((ACC,), jnp.float32),
            pltpu.VMEM((ACC,), jnp.float32),
            pltpu.VMEM((ACC,), jnp.float32),
            pltpu.VMEM((ACC,), jnp.float32),
            pltpu.VMEM((K, RH, W), jnp.float32),
            pltpu.VMEM((K, RH, W), jnp.float32),
            pltpu.VMEM((S,), jnp.float32),
            pltpu.SemaphoreType.DMA,
            pltpu.SemaphoreType.DMA,
        ],
    )(img4, idx2)


def kernel(img, index, max_index):
    del max_index
    b, f, h, w = img.shape
    idx2 = index.reshape(b, h * w).astype(jnp.int32)
    return _face_pool(img, idx2)


# speedup vs baseline: 1.0425x; 1.0082x over previous
import functools

import jax
import jax.numpy as jnp
from jax import lax
from jax.experimental import pallas as pl
from jax.experimental.pallas import tpu as pltpu
from jax.experimental.pallas import tpu_sc as plsc

B = 4
F = 192
HW = 224 * 224
S = 512
L = 16
NC, NS = 2, 16
NW = NC * NS
TPB = NW // B
FPT = F // TPB
K = 6
GRP = FPT // K
RH = 8
W = 224
CH = RH * W
NCH = 224 // RH
ACC = S * L + L


def _body(img_hbm, idx_hbm, out_hbm, addr_v,
          a0, a1, a2, a3, a4, a5,
          buf0, buf1, out_v0, out_v1, sem0, sem1, semo0, semo1):
    accs = (a0, a1, a2, a3, a4, a5)
    wid = lax.axis_index("s") * NC + lax.axis_index("c")
    b = wid // TPB
    f0 = (wid % TPB) * FPT
    lane = lax.broadcasted_iota(jnp.int32, (L,), 0)

    pltpu.sync_copy(idx_hbm.at[b], addr_v)

    def src(g, c):
        return img_hbm.at[b, pl.ds(f0 + g * K, K), pl.ds(c * RH, RH), :]

    def process(cur, ro, carry):
        def ldstep(i):
            r = i // (W // L)
            w0 = (i % (W // L)) * L
            v = addr_v[pl.ds(ro * W + i * L, L)]
            v = jnp.minimum(v, S)
            ad = jnp.where(v == 0, S * L + lane, (v - 1) * L + lane)
            vs = [cur[k, r, pl.ds(w0, L)] for k in range(K)]
            return ad, vs

        ad0, vs0 = ldstep(0)
        gs0 = [plsc.load_gather(accs[k], [ad0]) for k in range(K)]
        ms0 = tuple(jnp.maximum(g, v) for g, v in zip(gs0, vs0))

        def scat(i, st):
            cc, ad_p, ms_p = st
            ad, vs = ldstep(i)
            for k in range(K):
                plsc.store_scatter(accs[k], [ad_p], ms_p[k])
            gs = [plsc.load_gather(accs[k], [ad]) for k in range(K)]
            ms = tuple(jnp.maximum(g, v) for g, v in zip(gs, vs))
            return (cc, ad, ms)

        cc, ad_l, ms_l = lax.fori_loop(1, CH // L, scat, (carry, ad0, ms0))
        for k in range(K):
            plsc.store_scatter(accs[k], [ad_l], ms_l[k])
        return cc

    perm_idx = tuple(jnp.bitwise_xor(lane, d) for d in (8, 4, 2, 1))
    lane_bit = tuple((lane & d) == 0 for d in (8, 4, 2, 1))

    pltpu.async_copy(src(0, 0), buf0, sem0)
    pltpu.async_copy(src(0, 1), buf1, sem1)

    def group_body(g, carry):
        def zero(i, cc):
            z = jnp.zeros((L,), jnp.float32)
            for acc in accs:
                acc[pl.ds(i * L, L)] = z
            return cc

        carry = lax.fori_loop(0, ACC // L, zero, carry)

        def pair(c2, cc):
            c = c2 * 2
            pltpu.make_async_copy(src(g, c), buf0, sem0).wait()
            cc = process(buf0, c * RH, cc)
            pltpu.async_copy(src(g, c + 2), buf0, sem0)
            pltpu.make_async_copy(src(g, c + 1), buf1, sem1).wait()
            cc = process(buf1, (c + 1) * RH, cc)
            pltpu.async_copy(src(g, c + 3), buf1, sem1)
            return cc

        carry = lax.fori_loop(0, NCH // 2 - 1, pair, carry)

        gn = jnp.minimum(g + 1, GRP - 1)
        pltpu.make_async_copy(src(g, NCH - 2), buf0, sem0).wait()
        carry = process(buf0, (NCH - 2) * RH, carry)
        pltpu.async_copy(src(gn, 0), buf0, sem0)
        pltpu.make_async_copy(src(g, NCH - 1), buf1, sem1).wait()
        carry = process(buf1, (NCH - 1) * RH, carry)
        pltpu.async_copy(src(gn, 1), buf1, sem1)

        for k in range(K):
            ob, osem = (out_v0, semo0) if k % 2 == 0 else (out_v1, semo1)
            if k < 2:
                @pl.when(g > 0)
                def _wait():
                    pltpu.make_async_copy(ob, out_hbm.at[b, f0], osem).wait()
            else:
                pltpu.make_async_copy(ob, out_hbm.at[b, f0], osem).wait()

            def red(j, cc, *, acc=accs[k], ob=ob):
                rows = [acc[pl.ds(j * (L * L) + i * L, L)]
                        for i in range(L)]
                for lvl, d in enumerate((8, 4, 2, 1)):
                    half = len(rows) // 2
                    nxt = []
                    for i in range(half):
                        va, vb = rows[i], rows[i + half]
                        pa = va.at[perm_idx[lvl]].get(
                            mode="promise_in_bounds")
                        pb = vb.at[perm_idx[lvl]].get(
                            mode="promise_in_bounds")
                        nxt.append(jnp.where(lane_bit[lvl],
                                             jnp.maximum(va, pa),
                                             jnp.maximum(vb, pb)))
                    rows = nxt
                ob[pl.ds(j * L, L)] = rows[0]
                return cc

            carry = lax.fori_loop(0, S // L, red, carry)
            pltpu.async_copy(ob, out_hbm.at[b, f0 + g * K + k], osem)
        return carry

    lax.fori_loop(0, GRP, group_body, 0)
    pltpu.make_async_copy(src(0, 0), buf0, sem0).wait()
    pltpu.make_async_copy(src(0, 1), buf1, sem1).wait()
    pltpu.make_async_copy(out_v0, out_hbm.at[b, f0], semo0).wait()
    pltpu.make_async_copy(out_v1, out_hbm.at[b, f0], semo1).wait()


@jax.jit
def _face_pool(img4, idx2):
    mesh = plsc.VectorSubcoreMesh(core_axis_name="c", subcore_axis_name="s")
    return pl.kernel(
        _body,
        out_type=jax.ShapeDtypeStruct((B, F, S), jnp.float32),
        mesh=mesh,
        compiler_params=pltpu.CompilerParams(needs_layout_passes=False),
        scratch_types=[
            pltpu.VMEM((HW,), jnp.int32),
            pltpu.VMEM((ACC,), jnp.float32),
            pltpu.VMEM((ACC,), jnp.float32),
            pltpu.VMEM((ACC,), jnp.float32),
            pltpu.VMEM((ACC,), jnp.float32),
            pltpu.VMEM((ACC,), jnp.float32),
            pltpu.VMEM((ACC,), jnp.float32),
            pltpu.VMEM((K, RH, W), jnp.float32),
            pltpu.VMEM((K, RH, W), jnp.float32),
            pltpu.VMEM((S,), jnp.float32),
            pltpu.VMEM((S,), jnp.float32),
            pltpu.SemaphoreType.DMA,
            pltpu.SemaphoreType.DMA,
            pltpu.SemaphoreType.DMA,
            pltpu.SemaphoreType.DMA,
        ],
    )(img4, idx2)


def kernel(img, index, max_index):
    del max_index
    b, f, h, w = img.shape
    idx2 = index.reshape(b, h * w).astype(jnp.int32)
    return _face_pool(img, idx2)


# speedup vs baseline: 1.0804x; 1.0363x over previous
import functools

import jax
import jax.numpy as jnp
from jax import lax
from jax.experimental import pallas as pl
from jax.experimental.pallas import tpu as pltpu
from jax.experimental.pallas import tpu_sc as plsc

B = 4
F = 192
H = 224
W = 224
HW = H * W
S = 512
L = 16
NC, NS = 2, 16
NW = NC * NS
TPB = NW // B
FPT = F // TPB
K = 8
GRP = FPT // K
RH = 8
CH = RH * W
NCH = H // RH
ACC = S * L + L


def _body(img_hbm, idx_hbm, out_hbm,
          a0, a1, a2, a3, a4, a5, a6, a7,
          buf0, buf1, ibuf0, ibuf1, out_v0, out_v1,
          sem0, sem1, semi0, semi1, semo0, semo1):
    accs = (a0, a1, a2, a3, a4, a5, a6, a7)
    wid = lax.axis_index("s") * NC + lax.axis_index("c")
    b = wid // TPB
    f0 = (wid % TPB) * FPT
    lane = lax.broadcasted_iota(jnp.int32, (L,), 0)

    def src(g, c):
        return img_hbm.at[b, pl.ds(f0 + g * K, K), pl.ds(c * RH, RH), :]

    def isrc(c):
        return idx_hbm.at[b, pl.ds(c * RH, RH), :]

    def process(cur, icur, carry):
        def ldstep(i):
            r = i // (W // L)
            w0 = (i % (W // L)) * L
            v = icur[r, pl.ds(w0, L)]
            v = jnp.minimum(v, S)
            ad = jnp.where(v == 0, S * L + lane, (v - 1) * L + lane)
            vs = [cur[k, r, pl.ds(w0, L)] for k in range(K)]
            return ad, vs

        ad0, vs0 = ldstep(0)
        gs0 = [plsc.load_gather(accs[k], [ad0]) for k in range(K)]
        ms0 = tuple(jnp.maximum(g, v) for g, v in zip(gs0, vs0))

        def scat(i, st):
            cc, ad_p, ms_p = st
            ad, vs = ldstep(i)
            for k in range(K):
                plsc.store_scatter(accs[k], [ad_p], ms_p[k])
            gs = [plsc.load_gather(accs[k], [ad]) for k in range(K)]
            ms = tuple(jnp.maximum(g, v) for g, v in zip(gs, vs))
            return (cc, ad, ms)

        cc, ad_l, ms_l = lax.fori_loop(1, CH // L, scat, (carry, ad0, ms0))
        for k in range(K):
            plsc.store_scatter(accs[k], [ad_l], ms_l[k])
        return cc

    perm_idx = tuple(jnp.bitwise_xor(lane, d) for d in (8, 4, 2, 1))
    lane_bit = tuple((lane & d) == 0 for d in (8, 4, 2, 1))

    pltpu.async_copy(src(0, 0), buf0, sem0)
    pltpu.async_copy(isrc(0), ibuf0, semi0)
    pltpu.async_copy(src(0, 1), buf1, sem1)
    pltpu.async_copy(isrc(1), ibuf1, semi1)

    def group_body(g, carry):
        def zero(i, cc):
            z = jnp.zeros((L,), jnp.float32)
            for acc in accs:
                acc[pl.ds(i * L, L)] = z
            return cc

        carry = lax.fori_loop(0, ACC // L, zero, carry)

        def pair(c2, cc):
            c = c2 * 2
            pltpu.make_async_copy(src(g, c), buf0, sem0).wait()
            pltpu.make_async_copy(isrc(c), ibuf0, semi0).wait()
            cc = process(buf0, ibuf0, cc)
            pltpu.async_copy(src(g, c + 2), buf0, sem0)
            pltpu.async_copy(isrc(c + 2), ibuf0, semi0)
            pltpu.make_async_copy(src(g, c + 1), buf1, sem1).wait()
            pltpu.make_async_copy(isrc(c + 1), ibuf1, semi1).wait()
            cc = process(buf1, ibuf1, cc)
            pltpu.async_copy(src(g, c + 3), buf1, sem1)
            pltpu.async_copy(isrc(c + 3), ibuf1, semi1)
            return cc

        carry = lax.fori_loop(0, NCH // 2 - 1, pair, carry)

        gn = jnp.minimum(g + 1, GRP - 1)
        pltpu.make_async_copy(src(g, NCH - 2), buf0, sem0).wait()
        pltpu.make_async_copy(isrc(NCH - 2), ibuf0, semi0).wait()
        carry = process(buf0, ibuf0, carry)
        pltpu.async_copy(src(gn, 0), buf0, sem0)
        pltpu.async_copy(isrc(0), ibuf0, semi0)
        pltpu.make_async_copy(src(g, NCH - 1), buf1, sem1).wait()
        pltpu.make_async_copy(isrc(NCH - 1), ibuf1, semi1).wait()
        carry = process(buf1, ibuf1, carry)
        pltpu.async_copy(src(gn, 1), buf1, sem1)
        pltpu.async_copy(isrc(1), ibuf1, semi1)

        for k in range(K):
            ob, osem = (out_v0, semo0) if k % 2 == 0 else (out_v1, semo1)
            if k < 2:
                @pl.when(g > 0)
                def _wait():
                    pltpu.make_async_copy(ob, out_hbm.at[b, f0], osem).wait()
            else:
                pltpu.make_async_copy(ob, out_hbm.at[b, f0], osem).wait()

            def red(j, cc, *, acc=accs[k], ob=ob):
                rows = [acc[pl.ds(j * (L * L) + i * L, L)]
                        for i in range(L)]
                for lvl, d in enumerate((8, 4, 2, 1)):
                    half = len(rows) // 2
                    nxt = []
                    for i in range(half):
                        va, vb = rows[i], rows[i + half]
                        pa = va.at[perm_idx[lvl]].get(
                            mode="promise_in_bounds")
                        pb = vb.at[perm_idx[lvl]].get(
                            mode="promise_in_bounds")
                        nxt.append(jnp.where(lane_bit[lvl],
                                             jnp.maximum(va, pa),
                                             jnp.maximum(vb, pb)))
                    rows = nxt
                ob[pl.ds(j * L, L)] = rows[0]
                return cc

            carry = lax.fori_loop(0, S // L, red, carry)
            pltpu.async_copy(ob, out_hbm.at[b, f0 + g * K + k], osem)
        return carry

    lax.fori_loop(0, GRP, group_body, 0)
    pltpu.make_async_copy(src(0, 0), buf0, sem0).wait()
    pltpu.make_async_copy(isrc(0), ibuf0, semi0).wait()
    pltpu.make_async_copy(src(0, 1), buf1, sem1).wait()
    pltpu.make_async_copy(isrc(1), ibuf1, semi1).wait()
    pltpu.make_async_copy(out_v0, out_hbm.at[b, f0], semo0).wait()
    pltpu.make_async_copy(out_v1, out_hbm.at[b, f0], semo1).wait()


@jax.jit
def _face_pool(img4, idx3):
    mesh = plsc.VectorSubcoreMesh(core_axis_name="c", subcore_axis_name="s")
    return pl.kernel(
        _body,
        out_type=jax.ShapeDtypeStruct((B, F, S), jnp.float32),
        mesh=mesh,
        compiler_params=pltpu.CompilerParams(needs_layout_passes=False),
        scratch_types=[
            pltpu.VMEM((ACC,), jnp.float32),
            pltpu.VMEM((ACC,), jnp.float32),
            pltpu.VMEM((ACC,), jnp.float32),
            pltpu.VMEM((ACC,), jnp.float32),
            pltpu.VMEM((ACC,), jnp.float32),
            pltpu.VMEM((ACC,), jnp.float32),
            pltpu.VMEM((ACC,), jnp.float32),
            pltpu.VMEM((ACC,), jnp.float32),
            pltpu.VMEM((K, RH, W), jnp.float32),
            pltpu.VMEM((K, RH, W), jnp.float32),
            pltpu.VMEM((RH, W), jnp.int32),
            pltpu.VMEM((RH, W), jnp.int32),
            pltpu.VMEM((S,), jnp.float32),
            pltpu.VMEM((S,), jnp.float32),
            pltpu.SemaphoreType.DMA,
            pltpu.SemaphoreType.DMA,
            pltpu.SemaphoreType.DMA,
            pltpu.SemaphoreType.DMA,
            pltpu.SemaphoreType.DMA,
            pltpu.SemaphoreType.DMA,
        ],
    )(img4, idx3)


def kernel(img, index, max_index):
    del max_index
    return _face_pool(img, index.astype(jnp.int32))


# speedup vs baseline: 1.1114x; 1.0286x over previous
import functools

import jax
import jax.numpy as jnp
from jax import lax
from jax.experimental import pallas as pl
from jax.experimental.pallas import tpu as pltpu
from jax.experimental.pallas import tpu_sc as plsc

B = 4
F = 192
H = 224
W = 224
HW = H * W
S = 512
L = 16
NC, NS = 2, 16
NW = NC * NS
TPB = NW // B
FPT = F // TPB
K = 8
GRP = FPT // K
RH = 8
CH = RH * W
NCH = H // RH
ACC = S * L + L


def _body(img_hbm, idx_hbm, out_hbm,
          a0, a1, a2, a3, a4, a5, a6, a7,
          buf0, buf1, ibuf0, ibuf1, out_v0, out_v1,
          sem0, sem1, semi0, semi1, semo0, semo1):
    accs = (a0, a1, a2, a3, a4, a5, a6, a7)
    wid = lax.axis_index("s") * NC + lax.axis_index("c")
    b = wid // TPB
    f0 = (wid % TPB) * FPT
    lane = lax.broadcasted_iota(jnp.int32, (L,), 0)

    def src(g, c):
        return img_hbm.at[b, pl.ds(f0 + g * K, K), pl.ds(c * RH, RH), :]

    def isrc(c):
        return idx_hbm.at[b, pl.ds(c * RH, RH), :]

    def process(cur, icur, carry):
        def ldstep(i):
            r = i // (W // L)
            w0 = (i % (W // L)) * L
            v = icur[r, pl.ds(w0, L)]
            v = jnp.minimum(v, S)
            ad = jnp.where(v == 0, S * L + lane, (v - 1) * L + lane)
            vs = [cur[k, r, pl.ds(w0, L)] for k in range(K)]
            return ad, vs

        ad0, vs0 = ldstep(0)
        gs0 = [plsc.load_gather(accs[k], [ad0]) for k in range(K)]
        ms0 = tuple(jnp.maximum(g, v) for g, v in zip(gs0, vs0))

        def scat(i, st):
            cc, ad_p, ms_p = st
            ad, vs = ldstep(i)
            for k in range(K):
                plsc.store_scatter(accs[k], [ad_p], ms_p[k])
            gs = [plsc.load_gather(accs[k], [ad]) for k in range(K)]
            ms = tuple(jnp.maximum(g, v) for g, v in zip(gs, vs))
            return (cc, ad, ms)

        cc, ad_l, ms_l = lax.fori_loop(1, CH // L, scat, (carry, ad0, ms0))
        for k in range(K):
            plsc.store_scatter(accs[k], [ad_l], ms_l[k])
        return cc

    perm_idx = tuple(jnp.bitwise_xor(lane, d) for d in (8, 4, 2, 1))
    lane_bit = tuple((lane & d) == 0 for d in (8, 4, 2, 1))

    pltpu.async_copy(src(0, 0), buf0, sem0)
    pltpu.async_copy(isrc(0), ibuf0, semi0)
    pltpu.async_copy(src(0, 1), buf1, sem1)
    pltpu.async_copy(isrc(1), ibuf1, semi1)

    def group_body(g, carry):
        @functools.partial(plsc.parallel_loop, 0, ACC // L, unroll=4)
        def zero(i):
            z = jnp.zeros((L,), jnp.float32)
            for acc in accs:
                acc[pl.ds(i * L, L)] = z

        def pair(c2, cc):
            c = c2 * 2
            pltpu.make_async_copy(src(g, c), buf0, sem0).wait()
            pltpu.make_async_copy(isrc(c), ibuf0, semi0).wait()
            cc = process(buf0, ibuf0, cc)
            pltpu.async_copy(src(g, c + 2), buf0, sem0)
            pltpu.async_copy(isrc(c + 2), ibuf0, semi0)
            pltpu.make_async_copy(src(g, c + 1), buf1, sem1).wait()
            pltpu.make_async_copy(isrc(c + 1), ibuf1, semi1).wait()
            cc = process(buf1, ibuf1, cc)
            pltpu.async_copy(src(g, c + 3), buf1, sem1)
            pltpu.async_copy(isrc(c + 3), ibuf1, semi1)
            return cc

        carry = lax.fori_loop(0, NCH // 2 - 1, pair, carry)

        gn = jnp.minimum(g + 1, GRP - 1)
        pltpu.make_async_copy(src(g, NCH - 2), buf0, sem0).wait()
        pltpu.make_async_copy(isrc(NCH - 2), ibuf0, semi0).wait()
        carry = process(buf0, ibuf0, carry)
        pltpu.async_copy(src(gn, 0), buf0, sem0)
        pltpu.async_copy(isrc(0), ibuf0, semi0)
        pltpu.make_async_copy(src(g, NCH - 1), buf1, sem1).wait()
        pltpu.make_async_copy(isrc(NCH - 1), ibuf1, semi1).wait()
        carry = process(buf1, ibuf1, carry)
        pltpu.async_copy(src(gn, 1), buf1, sem1)
        pltpu.async_copy(isrc(1), ibuf1, semi1)

        for k in range(K):
            ob, osem = (out_v0, semo0) if k % 2 == 0 else (out_v1, semo1)
            if k < 2:
                @pl.when(g > 0)
                def _wait():
                    pltpu.make_async_copy(ob, out_hbm.at[b, f0], osem).wait()
            else:
                pltpu.make_async_copy(ob, out_hbm.at[b, f0], osem).wait()

            def red(j, cc, *, acc=accs[k], ob=ob):
                rows = [acc[pl.ds(j * (L * L) + i * L, L)]
                        for i in range(L)]
                for lvl, d in enumerate((8, 4, 2, 1)):
                    half = len(rows) // 2
                    nxt = []
                    for i in range(half):
                        va, vb = rows[i], rows[i + half]
                        pa = va.at[perm_idx[lvl]].get(
                            mode="promise_in_bounds")
                        pb = vb.at[perm_idx[lvl]].get(
                            mode="promise_in_bounds")
                        nxt.append(jnp.where(lane_bit[lvl],
                                             jnp.maximum(va, pa),
                                             jnp.maximum(vb, pb)))
                    rows = nxt
                ob[pl.ds(j * L, L)] = rows[0]
                return cc

            carry = lax.fori_loop(0, S // L, red, carry)
            pltpu.async_copy(ob, out_hbm.at[b, f0 + g * K + k], osem)
        return carry

    lax.fori_loop(0, GRP, group_body, 0)
    pltpu.make_async_copy(src(0, 0), buf0, sem0).wait()
    pltpu.make_async_copy(isrc(0), ibuf0, semi0).wait()
    pltpu.make_async_copy(src(0, 1), buf1, sem1).wait()
    pltpu.make_async_copy(isrc(1), ibuf1, semi1).wait()
    pltpu.make_async_copy(out_v0, out_hbm.at[b, f0], semo0).wait()
    pltpu.make_async_copy(out_v1, out_hbm.at[b, f0], semo1).wait()


@jax.jit
def _face_pool(img4, idx3):
    mesh = plsc.VectorSubcoreMesh(core_axis_name="c", subcore_axis_name="s")
    return pl.kernel(
        _body,
        out_type=jax.ShapeDtypeStruct((B, F, S), jnp.float32),
        mesh=mesh,
        compiler_params=pltpu.CompilerParams(needs_layout_passes=False),
        scratch_types=[
            pltpu.VMEM((ACC,), jnp.float32),
            pltpu.VMEM((ACC,), jnp.float32),
            pltpu.VMEM((ACC,), jnp.float32),
            pltpu.VMEM((ACC,), jnp.float32),
            pltpu.VMEM((ACC,), jnp.float32),
            pltpu.VMEM((ACC,), jnp.float32),
            pltpu.VMEM((ACC,), jnp.float32),
            pltpu.VMEM((ACC,), jnp.float32),
            pltpu.VMEM((K, RH, W), jnp.float32),
            pltpu.VMEM((K, RH, W), jnp.float32),
            pltpu.VMEM((RH, W), jnp.int32),
            pltpu.VMEM((RH, W), jnp.int32),
            pltpu.VMEM((S,), jnp.float32),
            pltpu.VMEM((S,), jnp.float32),
            pltpu.SemaphoreType.DMA,
            pltpu.SemaphoreType.DMA,
            pltpu.SemaphoreType.DMA,
            pltpu.SemaphoreType.DMA,
            pltpu.SemaphoreType.DMA,
            pltpu.SemaphoreType.DMA,
        ],
    )(img4, idx3)


def kernel(img, index, max_index):
    del max_index
    return _face_pool(img, index.astype(jnp.int32))


# speedup vs baseline: 1.1197x; 1.0075x over previous
import functools

import jax
import jax.numpy as jnp
from jax import lax
from jax.experimental import pallas as pl
from jax.experimental.pallas import tpu as pltpu
from jax.experimental.pallas import tpu_sc as plsc

B = 4
F = 192
H = 224
W = 224
HW = H * W
S = 512
L = 16
NC, NS = 2, 16
NW = NC * NS
TPB = NW // B
FPT = F // TPB
K = 8
GRP = FPT // K
RH = 8
CH = RH * W
NCH = H // RH
ACC = S * L + L


def _body(img_hbm, idx_hbm, out_hbm,
          a0, a1, a2, a3, a4, a5, a6, a7,
          buf0, buf1, ibuf0, ibuf1, out_v0, out_v1,
          sem0, sem1, semi0, semi1, semo0, semo1):
    accs = (a0, a1, a2, a3, a4, a5, a6, a7)
    wid = lax.axis_index("s") * NC + lax.axis_index("c")
    b = wid // TPB
    f0 = (wid % TPB) * FPT
    lane = lax.broadcasted_iota(jnp.int32, (L,), 0)

    def src(g, c):
        return img_hbm.at[b, pl.ds(f0 + g * K, K), pl.ds(c * RH, RH), :]

    def isrc(c):
        return idx_hbm.at[b, pl.ds(c * RH, RH), :]

    def process(cur, icur, carry):
        def ldstep(i):
            r = i // (W // L)
            w0 = (i % (W // L)) * L
            v = icur[r, pl.ds(w0, L)]
            v = jnp.minimum(v, S)
            ad = jnp.where(v == 0, S * L + lane, (v - 1) * L + lane)
            vs = [cur[k, r, pl.ds(w0, L)] for k in range(K)]
            return ad, vs

        def stage(i, ad_p, ms_p):
            ad, vs = ldstep(i)
            for k in range(K):
                plsc.store_scatter(accs[k], [ad_p], ms_p[k])
            gs = [plsc.load_gather(accs[k], [ad]) for k in range(K)]
            ms = tuple(jnp.maximum(g, v) for g, v in zip(gs, vs))
            return ad, ms

        ad0, vs0 = ldstep(0)
        gs0 = [plsc.load_gather(accs[k], [ad0]) for k in range(K)]
        ms0 = tuple(jnp.maximum(g, v) for g, v in zip(gs0, vs0))

        def scat(i, st):
            cc, ad_p, ms_p = st
            ad_a, ms_a = stage(2 * i - 1, ad_p, ms_p)
            ad_b, ms_b = stage(2 * i, ad_a, ms_a)
            return (cc, ad_b, ms_b)

        cc, ad_l, ms_l = lax.fori_loop(1, CH // (2 * L), scat,
                                       (carry, ad0, ms0))
        ad_l, ms_l = stage(CH // L - 1, ad_l, ms_l)
        for k in range(K):
            plsc.store_scatter(accs[k], [ad_l], ms_l[k])
        return cc

    perm_idx = tuple(jnp.bitwise_xor(lane, d) for d in (8, 4, 2, 1))
    lane_bit = tuple((lane & d) == 0 for d in (8, 4, 2, 1))

    pltpu.async_copy(src(0, 0), buf0, sem0)
    pltpu.async_copy(isrc(0), ibuf0, semi0)
    pltpu.async_copy(src(0, 1), buf1, sem1)
    pltpu.async_copy(isrc(1), ibuf1, semi1)

    def group_body(g, carry):
        @functools.partial(plsc.parallel_loop, 0, ACC // L, unroll=4)
        def zero(i):
            z = jnp.zeros((L,), jnp.float32)
            for acc in accs:
                acc[pl.ds(i * L, L)] = z

        def pair(c2, cc):
            c = c2 * 2
            pltpu.make_async_copy(src(g, c), buf0, sem0).wait()
            pltpu.make_async_copy(isrc(c), ibuf0, semi0).wait()
            cc = process(buf0, ibuf0, cc)
            pltpu.async_copy(src(g, c + 2), buf0, sem0)
            pltpu.async_copy(isrc(c + 2), ibuf0, semi0)
            pltpu.make_async_copy(src(g, c + 1), buf1, sem1).wait()
            pltpu.make_async_copy(isrc(c + 1), ibuf1, semi1).wait()
            cc = process(buf1, ibuf1, cc)
            pltpu.async_copy(src(g, c + 3), buf1, sem1)
            pltpu.async_copy(isrc(c + 3), ibuf1, semi1)
            return cc

        carry = lax.fori_loop(0, NCH // 2 - 1, pair, carry)

        gn = jnp.minimum(g + 1, GRP - 1)
        pltpu.make_async_copy(src(g, NCH - 2), buf0, sem0).wait()
        pltpu.make_async_copy(isrc(NCH - 2), ibuf0, semi0).wait()
        carry = process(buf0, ibuf0, carry)
        pltpu.async_copy(src(gn, 0), buf0, sem0)
        pltpu.async_copy(isrc(0), ibuf0, semi0)
        pltpu.make_async_copy(src(g, NCH - 1), buf1, sem1).wait()
        pltpu.make_async_copy(isrc(NCH - 1), ibuf1, semi1).wait()
        carry = process(buf1, ibuf1, carry)
        pltpu.async_copy(src(gn, 1), buf1, sem1)
        pltpu.async_copy(isrc(1), ibuf1, semi1)

        for k in range(K):
            ob, osem = (out_v0, semo0) if k % 2 == 0 else (out_v1, semo1)
            if k < 2:
                @pl.when(g > 0)
                def _wait():
                    pltpu.make_async_copy(ob, out_hbm.at[b, f0], osem).wait()
            else:
                pltpu.make_async_copy(ob, out_hbm.at[b, f0], osem).wait()

            def red(j, cc, *, acc=accs[k], ob=ob):
                rows = [acc[pl.ds(j * (L * L) + i * L, L)]
                        for i in range(L)]
                for lvl, d in enumerate((8, 4, 2, 1)):
                    half = len(rows) // 2
                    nxt = []
                    for i in range(half):
                        va, vb = rows[i], rows[i + half]
                        pa = va.at[perm_idx[lvl]].get(
                            mode="promise_in_bounds")
                        pb = vb.at[perm_idx[lvl]].get(
                            mode="promise_in_bounds")
                        nxt.append(jnp.where(lane_bit[lvl],
                                             jnp.maximum(va, pa),
                                             jnp.maximum(vb, pb)))
                    rows = nxt
                ob[pl.ds(j * L, L)] = rows[0]
                return cc

            carry = lax.fori_loop(0, S // L, red, carry)
            pltpu.async_copy(ob, out_hbm.at[b, f0 + g * K + k], osem)
        return carry

    lax.fori_loop(0, GRP, group_body, 0)
    pltpu.make_async_copy(src(0, 0), buf0, sem0).wait()
    pltpu.make_async_copy(isrc(0), ibuf0, semi0).wait()
    pltpu.make_async_copy(src(0, 1), buf1, sem1).wait()
    pltpu.make_async_copy(isrc(1), ibuf1, semi1).wait()
    pltpu.make_async_copy(out_v0, out_hbm.at[b, f0], semo0).wait()
    pltpu.make_async_copy(out_v1, out_hbm.at[b, f0], semo1).wait()


@jax.jit
def _face_pool(img4, idx3):
    mesh = plsc.VectorSubcoreMesh(core_axis_name="c", subcore_axis_name="s")
    return pl.kernel(
        _body,
        out_type=jax.ShapeDtypeStruct((B, F, S), jnp.float32),
        mesh=mesh,
        compiler_params=pltpu.CompilerParams(needs_layout_passes=False),
        scratch_types=[
            pltpu.VMEM((ACC,), jnp.float32),
            pltpu.VMEM((ACC,), jnp.float32),
            pltpu.VMEM((ACC,), jnp.float32),
            pltpu.VMEM((ACC,), jnp.float32),
            pltpu.VMEM((ACC,), jnp.float32),
            pltpu.VMEM((ACC,), jnp.float32),
            pltpu.VMEM((ACC,), jnp.float32),
            pltpu.VMEM((ACC,), jnp.float32),
            pltpu.VMEM((K, RH, W), jnp.float32),
            pltpu.VMEM((K, RH, W), jnp.float32),
            pltpu.VMEM((RH, W), jnp.int32),
            pltpu.VMEM((RH, W), jnp.int32),
            pltpu.VMEM((S,), jnp.float32),
            pltpu.VMEM((S,), jnp.float32),
            pltpu.SemaphoreType.DMA,
            pltpu.SemaphoreType.DMA,
            pltpu.SemaphoreType.DMA,
            pltpu.SemaphoreType.DMA,
            pltpu.SemaphoreType.DMA,
            pltpu.SemaphoreType.DMA,
        ],
    )(img4, idx3)


def kernel(img, index, max_index):
    del max_index
    return _face_pool(img, index.astype(jnp.int32))
